# baseline (speedup 1.0000x reference)
import functools

import jax
import jax.numpy as jnp
from jax import lax
from jax.experimental import pallas as pl
from jax.experimental.pallas import tpu as pltpu
from jax.experimental.pallas import tpu_sc as plsc

BATCH = 16384
DIM = 32
NW = 32
PER_W = BATCH // NW
CHUNK = 128
NCHUNK = PER_W // CHUNK
GROUPS = PER_W // 16


def _sc_body(uidx_hbm, pidx_hbm, nidx_hbm, utab, itab, btab, out_hbm,
             uidx_v, pidx_v, nidx_v, urows, prows, nrows,
             bp, bn, outp, outn, sem):
    w = lax.axis_index("s") * 2 + lax.axis_index("c")

    pltpu.sync_copy(uidx_hbm.at[w], uidx_v)
    pltpu.sync_copy(pidx_hbm.at[w], pidx_v)
    pltpu.sync_copy(nidx_hbm.at[w], nidx_v)

    copies = []
    for j in range(NCHUNK):
        sl = pl.ds(j * CHUNK, CHUNK)
        copies.append(pltpu.async_copy(utab.at[uidx_v.at[j]], urows.at[sl], sem))
        copies.append(pltpu.async_copy(itab.at[pidx_v.at[j]], prows.at[sl], sem))
        copies.append(pltpu.async_copy(itab.at[nidx_v.at[j]], nrows.at[sl], sem))
        copies.append(pltpu.async_copy(btab.at[pidx_v.at[j]], bp.at[sl], sem))
        copies.append(pltpu.async_copy(btab.at[nidx_v.at[j]], bn.at[sl], sem))
    for c in copies:
        c.wait()

    lane = lax.iota(jnp.int32, 16)

    def group(g, _):
        rid = lane + g * 16
        accp = bp[pl.ds(g * 16, 16)]
        accn = bn[pl.ds(g * 16, 16)]
        for d in range(DIM):
            cd = jnp.full((16,), d, jnp.int32)
            uv = plsc.load_gather(urows, [rid, cd])
            pv = plsc.load_gather(prows, [rid, cd])
            nv = plsc.load_gather(nrows, [rid, cd])
            accp = accp + uv * pv
            accn = accn + uv * nv
        outp[pl.ds(g * 16, 16)] = accp
        outn[pl.ds(g * 16, 16)] = accn
        return 0

    lax.fori_loop(0, GROUPS, group, 0)

    pltpu.sync_copy(outp, out_hbm.at[pl.ds(w * PER_W, PER_W)])
    pltpu.sync_copy(outn, out_hbm.at[pl.ds(w * PER_W + BATCH, PER_W)])


_sc_call = functools.partial(
    pl.kernel,
    mesh=plsc.VectorSubcoreMesh(core_axis_name="c", subcore_axis_name="s"),
    out_type=jax.ShapeDtypeStruct((2 * BATCH,), jnp.float32),
    compiler_params=pltpu.CompilerParams(
        use_tc_tiling_on_sc=False, needs_layout_passes=False
    ),
    scratch_types=[
        pltpu.VMEM((NCHUNK, CHUNK), jnp.int32),
        pltpu.VMEM((NCHUNK, CHUNK), jnp.int32),
        pltpu.VMEM((NCHUNK, CHUNK), jnp.int32),
        pltpu.VMEM((PER_W, DIM), jnp.float32),
        pltpu.VMEM((PER_W, DIM), jnp.float32),
        pltpu.VMEM((PER_W, DIM), jnp.float32),
        pltpu.VMEM((PER_W,), jnp.float32),
        pltpu.VMEM((PER_W,), jnp.float32),
        pltpu.VMEM((PER_W,), jnp.float32),
        pltpu.VMEM((PER_W,), jnp.float32),
        pltpu.SemaphoreType.DMA,
    ],
)(_sc_body)


def kernel(batch_data, u_embedding, i_embedding, i_bias):
    idx = batch_data.astype(jnp.int32)
    uidx = idx[:, 0].reshape(NW, NCHUNK, CHUNK)
    pidx = idx[:, 1].reshape(NW, NCHUNK, CHUNK)
    nidx = idx[:, 2].reshape(NW, NCHUNK, CHUNK)
    logits = _sc_call(uidx, pidx, nidx, u_embedding, i_embedding,
                      i_bias.reshape(-1))
    labels = jnp.concatenate([
        jnp.ones((BATCH,), dtype=jnp.float32),
        jnp.zeros((BATCH,), dtype=jnp.float32),
    ])
    return (logits.reshape(2 * BATCH, 1), labels)
